# baseline (speedup 1.0000x reference)
import jax
import jax.numpy as jnp
from jax.experimental import pallas as pl

K = 300


def _sig_body(x_ref, o_ref):
    o_ref[...] = jax.nn.sigmoid(x_ref[...])


def kernel(pred_logits, pred_boxes, target_sizes):
    B, Q, C = pred_logits.shape
    x = pred_logits.reshape(B, Q * C)
    prob = pl.pallas_call(
        _sig_body,
        out_shape=jax.ShapeDtypeStruct(x.shape, x.dtype),
    )(x)
    topk_values, topk_indexes = jax.lax.top_k(prob, K)
    scores = topk_values
    topk_boxes = topk_indexes // C
    labels = topk_indexes % C
    cx, cy, w, h = jnp.split(pred_boxes, 4, axis=-1)
    boxes = jnp.concatenate(
        [cx - 0.5 * w, cy - 0.5 * h, cx + 0.5 * w, cy + 0.5 * h], axis=-1
    )
    boxes = jnp.take_along_axis(boxes, topk_boxes[..., None], axis=1)
    img_h = target_sizes[:, 0].astype(jnp.float32)
    img_w = target_sizes[:, 1].astype(jnp.float32)
    scale_fct = jnp.stack([img_w, img_h, img_w, img_h], axis=1)
    boxes = boxes * scale_fct[:, None, :]
    return (scores, labels, boxes)


# speedup vs baseline: 4.3729x; 4.3729x over previous
import functools

import jax
import jax.numpy as jnp
from jax import lax
from jax.experimental import pallas as pl
from jax.experimental.pallas import tpu as pltpu
from jax.experimental.pallas import tpu_sc as plsc

B, Q, C = 32, 900, 91
N = Q * C
NPAD = 81920
K = 300
KPAD = 304
NVEC = NPAD // 16
UNROLL = 4
HIST = 8192


def _sig_body(x_ref, o_ref):
    o_ref[...] = jax.nn.sigmoid(x_ref[...])


def _scan_hist(hist_ref, nbuckets, need):
    nchunks = nbuckets // 16
    lanes = lax.iota(jnp.int32, 16)

    def body(k, carry):
        run, found, b, s_above = carry
        chunk = nchunks - 1 - k
        c = hist_ref[pl.ds(chunk * 16, 16)]
        cr = lax.rev(c, (0,))
        cs = plsc.cumsum(cr)
        s = cs + run
        crossed = s >= need
        n = plsc.all_reduce_ffs(crossed)
        n0 = jnp.min(n)
        hit = jnp.logical_and(jnp.logical_not(found), n0 < 16)
        s_at = jnp.max(jnp.where(lanes == n0, s, 0))
        cr_at = jnp.max(jnp.where(lanes == n0, cr, 0))
        b_new = jnp.where(hit, chunk * 16 + 15 - n0, b)
        s_above_new = jnp.where(hit, s_at - cr_at, s_above)
        found_new = jnp.logical_or(found, n0 < 16)
        run_new = run + jnp.max(cs)
        return run_new, found_new, b_new, s_above_new

    init = (jnp.int32(0), jnp.bool_(False), jnp.int32(0), jnp.int32(0))
    _, _, b, s_above = lax.fori_loop(0, nchunks, body, init)
    return b, s_above


def _zero(ref, nwords):
    zeros = jnp.zeros((16,), jnp.int32)

    def body(k, _):
        ref[pl.ds(k * 16, 16)] = zeros
        return 0

    lax.fori_loop(0, nwords // 16, body, 0)


def _sc_body(p_hbm, boxes_hbm, scale_hbm,
             scores_hbm, labels_hbm, boxes_out_hbm,
             p_v, boxes_v, scale_v, hist_v,
             cand_u, cand_idx, out_pos,
             scores_v, labels_v, boxes_ov):
    nc = 2
    wid = lax.axis_index("s") * nc + lax.axis_index("c")
    lanes = lax.iota(jnp.int32, 16)

    pltpu.sync_copy(p_hbm.at[wid], p_v)
    pltpu.sync_copy(boxes_hbm.at[wid], boxes_v)
    pltpu.sync_copy(scale_hbm.at[wid], scale_v)

    _zero(hist_v, HIST)

    def r0_body(j, _):
        for k in range(UNROLL):
            u = p_v[pl.ds((j * UNROLL + k) * 16, 16)]
            b = jnp.right_shift(u, 17)
            cnt, last = plsc.scan_count(b)
            plsc.addupdate_scatter(hist_v, [b], cnt, mask=last)
        return 0

    lax.fori_loop(0, NVEC // UNROLL, r0_body, 0)
    b0, gt0 = _scan_hist(hist_v, HIST, K)

    _zero(hist_v, 512)
    need1 = K - gt0

    def r1_body(j, _):
        for k in range(UNROLL):
            u = p_v[pl.ds((j * UNROLL + k) * 16, 16)]
            pm = jnp.right_shift(u, 17) == b0
            d = jnp.right_shift(u, 8) & 0x1FF
            cnt, last = plsc.scan_count(d, mask=pm)
            plsc.addupdate_scatter(hist_v, [d], cnt, mask=last)
        return 0

    lax.fori_loop(0, NVEC // UNROLL, r1_body, 0)
    b1, gt1 = _scan_hist(hist_v, 512, need1)
    p2 = (b0 << 9) | b1

    _zero(hist_v, 256)
    need2 = need1 - gt1

    def r2_body(j, _):
        for k in range(UNROLL):
            u = p_v[pl.ds((j * UNROLL + k) * 16, 16)]
            pm = jnp.right_shift(u, 8) == p2
            d = u & 0xFF
            cnt, last = plsc.scan_count(d, mask=pm)
            plsc.addupdate_scatter(hist_v, [d], cnt, mask=last)
        return 0

    lax.fori_loop(0, NVEC // UNROLL, r2_body, 0)
    b2, gt2 = _scan_hist(hist_v, 256, need2)
    t = (p2 << 8) | b2
    cnt_gt = gt0 + gt1 + gt2
    need_eq = K - cnt_gt

    neg1 = jnp.full((16,), -1, jnp.int32)
    for j in range((KPAD + 16) // 16):
        cand_u[pl.ds(j * 16, 16)] = neg1

    def c_body(j, carry):
        sel_cnt, eq_cnt = carry
        for k in range(UNROLL):
            base = (j * UNROLL + k) * 16
            u = p_v[pl.ds(base, 16)]
            idxv = lanes + base
            msel = u > t
            meq = u == t
            plsc.store_compressed(cand_u.at[pl.ds(sel_cnt, 16)], u, mask=msel)
            plsc.store_compressed(cand_idx.at[pl.ds(sel_cnt, 16)], idxv,
                                  mask=msel)
            nsel = jnp.min(plsc.all_reduce_population_count(msel))
            sel_cnt = sel_cnt + nsel

            eq_open = eq_cnt < need_eq

            @pl.when(eq_open)
            def _():
                off = cnt_gt + eq_cnt
                plsc.store_compressed(cand_u.at[pl.ds(off, 16)], u, mask=meq)
                plsc.store_compressed(cand_idx.at[pl.ds(off, 16)], idxv,
                                      mask=meq)

            neq = jnp.min(plsc.all_reduce_population_count(meq))
            eq_cnt = jnp.where(eq_open, eq_cnt + neq, eq_cnt)
        return sel_cnt, eq_cnt

    lax.fori_loop(0, NVEC // UNROLL, c_body,
                  (jnp.int32(0), jnp.int32(0)))

    lane0 = lanes == 0
    zeros16 = jnp.zeros((16,), jnp.int32)

    def s_body(r, _):
        cu = cand_u[pl.ds(0, 16)]
        cp = lanes
        for jj in range(1, KPAD // 16):
            uv = cand_u[pl.ds(jj * 16, 16)]
            pv = lanes + jj * 16
            take = uv > cu
            cu = jnp.where(take, uv, cu)
            cp = jnp.where(take, pv, cp)
        m = jnp.max(cu)
        posm = jnp.where(cu == m, cp, jnp.int32(100000))
        pos = jnp.min(posm)
        posv = jnp.broadcast_to(pos, (16,))
        plsc.store_scatter(out_pos, [zeros16 + r], posv, mask=lane0)
        plsc.store_scatter(cand_u, [posv], jnp.full((16,), -2, jnp.int32),
                           mask=lane0)
        return 0

    out_pos[pl.ds(0, 16)] = zeros16
    out_pos[pl.ds(KPAD - 16, 16)] = zeros16
    lax.fori_loop(0, K, s_body, 0)

    wv = scale_v[pl.ds(0, 16)]
    hv = scale_v[pl.ds(16, 16)]
    half = jnp.float32(0.5)
    for jj in range(KPAD // 16):
        pv = out_pos[pl.ds(jj * 16, 16)]
        gidx = plsc.load_gather(cand_idx, [pv])
        score = plsc.load_gather(p_v, [gidx])
        q = gidx // C
        lab = gidx - q * C
        q4 = q * 4
        cx = plsc.load_gather(boxes_v, [q4])
        cy = plsc.load_gather(boxes_v, [q4 + 1])
        w = plsc.load_gather(boxes_v, [q4 + 2])
        h = plsc.load_gather(boxes_v, [q4 + 3])
        hw = half * w
        hh = half * h
        scores_v[pl.ds(jj * 16, 16)] = score
        labels_v[pl.ds(jj * 16, 16)] = lab
        boxes_ov[pl.ds(0 * KPAD + jj * 16, 16)] = (cx - hw) * wv
        boxes_ov[pl.ds(1 * KPAD + jj * 16, 16)] = (cy - hh) * hv
        boxes_ov[pl.ds(2 * KPAD + jj * 16, 16)] = (cx + hw) * wv
        boxes_ov[pl.ds(3 * KPAD + jj * 16, 16)] = (cy + hh) * hv

    pltpu.sync_copy(scores_v, scores_hbm.at[wid])
    pltpu.sync_copy(labels_v, labels_hbm.at[wid])
    pltpu.sync_copy(boxes_ov, boxes_out_hbm.at[wid])


def _sc_topk(p, boxes_flat, scale):
    mesh = plsc.VectorSubcoreMesh(core_axis_name="c", subcore_axis_name="s")
    f = pl.kernel(
        _sc_body,
        out_type=(
            jax.ShapeDtypeStruct((B, KPAD), jnp.int32),
            jax.ShapeDtypeStruct((B, KPAD), jnp.int32),
            jax.ShapeDtypeStruct((B, 4 * KPAD), jnp.float32),
        ),
        mesh=mesh,
        compiler_params=pltpu.CompilerParams(needs_layout_passes=False),
        scratch_types=[
            pltpu.VMEM((NPAD,), jnp.int32),
            pltpu.VMEM((4 * Q,), jnp.float32),
            pltpu.VMEM((32,), jnp.float32),
            pltpu.VMEM((HIST,), jnp.int32),
            pltpu.VMEM((KPAD + 16,), jnp.int32),
            pltpu.VMEM((KPAD + 16,), jnp.int32),
            pltpu.VMEM((KPAD,), jnp.int32),
            pltpu.VMEM((KPAD,), jnp.int32),
            pltpu.VMEM((KPAD,), jnp.int32),
            pltpu.VMEM((4 * KPAD,), jnp.float32),
        ],
    )
    return f(p, boxes_flat, scale)


def kernel(pred_logits, pred_boxes, target_sizes):
    x = pred_logits.reshape(B, N)
    xp = jnp.pad(x, ((0, 0), (0, NPAD - N)), constant_values=-1e38)
    p = pl.pallas_call(
        _sig_body,
        out_shape=jax.ShapeDtypeStruct((B, NPAD), jnp.float32),
    )(xp)
    boxes_flat = pred_boxes.reshape(B, 4 * Q)
    ts = target_sizes.astype(jnp.float32)
    img_h = ts[:, 0]
    img_w = ts[:, 1]
    scale = jnp.concatenate(
        [jnp.broadcast_to(img_w[:, None], (B, 16)),
         jnp.broadcast_to(img_h[:, None], (B, 16))], axis=1)
    p_bits = jax.lax.bitcast_convert_type(p, jnp.int32)
    scores_p, labels_p, boxes_p = _sc_topk(p_bits, boxes_flat, scale)
    scores = jax.lax.bitcast_convert_type(scores_p[:, :K], jnp.float32)
    labels = labels_p[:, :K]
    boxes = boxes_p.reshape(B, 4, KPAD).transpose(0, 2, 1)[:, :K, :]
    return (scores, labels, boxes)


# speedup vs baseline: 6.0641x; 1.3867x over previous
import functools

import jax
import jax.numpy as jnp
from jax import lax
from jax.experimental import pallas as pl
from jax.experimental.pallas import tpu as pltpu
from jax.experimental.pallas import tpu_sc as plsc

B, Q, C = 32, 900, 91
N = Q * C
NPAD = 81920
K = 300
KPAD = 304
NVEC = NPAD // 16
UNROLL = 4
HIST = 8192


def _sig_body(x_ref, o_ref):
    o_ref[...] = jax.nn.sigmoid(x_ref[...])


def _scan_hist(hist_ref, nbuckets, need):
    nchunks = nbuckets // 16
    lanes = lax.iota(jnp.int32, 16)

    def body(k, carry):
        run, found, b, s_above = carry
        chunk = nchunks - 1 - k
        c = hist_ref[pl.ds(chunk * 16, 16)]
        cr = lax.rev(c, (0,))
        cs = plsc.cumsum(cr)
        s = cs + run
        crossed = s >= need
        n = plsc.all_reduce_ffs(crossed)
        n0 = jnp.min(n)
        hit = jnp.logical_and(jnp.logical_not(found), n0 < 16)
        s_at = jnp.max(jnp.where(lanes == n0, s, 0))
        cr_at = jnp.max(jnp.where(lanes == n0, cr, 0))
        b_new = jnp.where(hit, chunk * 16 + 15 - n0, b)
        s_above_new = jnp.where(hit, s_at - cr_at, s_above)
        found_new = jnp.logical_or(found, n0 < 16)
        run_new = run + jnp.max(cs)
        return run_new, found_new, b_new, s_above_new

    init = (jnp.int32(0), jnp.bool_(False), jnp.int32(0), jnp.int32(0))
    _, _, b, s_above = lax.fori_loop(0, nchunks, body, init)
    return b, s_above


def _zero(ref, nwords):
    zeros = jnp.zeros((16,), jnp.int32)

    def body(k, _):
        ref[pl.ds(k * 16, 16)] = zeros
        return 0

    lax.fori_loop(0, nwords // 16, body, 0)


def _sc_body(p_hbm, boxes_hbm, scale_hbm,
             scores_hbm, labels_hbm, boxes_out_hbm,
             p_v, boxes_v, scale_v, hist_v,
             cand_u, cand_idx, out_pos,
             scores_v, labels_v, boxes_ov):
    nc = 2
    wid = lax.axis_index("s") * nc + lax.axis_index("c")
    lanes = lax.iota(jnp.int32, 16)
    ones16 = jnp.ones((16,), jnp.int32)

    pltpu.sync_copy(p_hbm.at[wid], p_v)
    pltpu.sync_copy(boxes_hbm.at[wid], boxes_v)
    pltpu.sync_copy(scale_hbm.at[wid], scale_v)

    _zero(hist_v, HIST)

    def r0_body(j, _):
        for k in range(UNROLL):
            u = p_v[pl.ds((j * UNROLL + k) * 16, 16)]
            b = jnp.right_shift(u, 17)
            plsc.addupdate_scatter(hist_v, [b], ones16)
        return 0

    lax.fori_loop(0, NVEC // UNROLL, r0_body, 0)
    b0, gt0 = _scan_hist(hist_v, HIST, K)

    _zero(hist_v, 512)
    need1 = K - gt0

    def r1_body(j, _):
        for k in range(UNROLL):
            u = p_v[pl.ds((j * UNROLL + k) * 16, 16)]
            pm = jnp.right_shift(u, 17) == b0
            d = jnp.right_shift(u, 8) & 0x1FF
            plsc.addupdate_scatter(hist_v, [d], ones16, mask=pm)
        return 0

    lax.fori_loop(0, NVEC // UNROLL, r1_body, 0)
    b1, gt1 = _scan_hist(hist_v, 512, need1)
    p2 = (b0 << 9) | b1

    _zero(hist_v, 256)
    need2 = need1 - gt1

    def r2_body(j, _):
        for k in range(UNROLL):
            u = p_v[pl.ds((j * UNROLL + k) * 16, 16)]
            pm = jnp.right_shift(u, 8) == p2
            d = u & 0xFF
            plsc.addupdate_scatter(hist_v, [d], ones16, mask=pm)
        return 0

    lax.fori_loop(0, NVEC // UNROLL, r2_body, 0)
    b2, gt2 = _scan_hist(hist_v, 256, need2)
    t = (p2 << 8) | b2
    cnt_gt = gt0 + gt1 + gt2
    need_eq = K - cnt_gt

    neg1 = jnp.full((16,), -1, jnp.int32)
    for j in range((KPAD + 16) // 16):
        cand_u[pl.ds(j * 16, 16)] = neg1

    def c_body(j, carry):
        sel_cnt, eq_cnt = carry
        for k in range(UNROLL):
            base = (j * UNROLL + k) * 16
            u = p_v[pl.ds(base, 16)]
            idxv = lanes + base
            msel = u > t
            meq = u == t
            plsc.store_compressed(cand_u.at[pl.ds(sel_cnt, 16)], u, mask=msel)
            plsc.store_compressed(cand_idx.at[pl.ds(sel_cnt, 16)], idxv,
                                  mask=msel)
            nsel = jnp.min(plsc.all_reduce_population_count(msel))
            sel_cnt = sel_cnt + nsel

            eq_open = eq_cnt < need_eq

            @pl.when(eq_open)
            def _():
                off = cnt_gt + eq_cnt
                plsc.store_compressed(cand_u.at[pl.ds(off, 16)], u, mask=meq)
                plsc.store_compressed(cand_idx.at[pl.ds(off, 16)], idxv,
                                      mask=meq)

            neq = jnp.min(plsc.all_reduce_population_count(meq))
            eq_cnt = jnp.where(eq_open, eq_cnt + neq, eq_cnt)
        return sel_cnt, eq_cnt

    lax.fori_loop(0, NVEC // UNROLL, c_body,
                  (jnp.int32(0), jnp.int32(0)))

    lane0 = lanes == 0
    zeros16 = jnp.zeros((16,), jnp.int32)

    def s_body(r, _):
        cu = cand_u[pl.ds(0, 16)]
        cp = lanes
        for jj in range(1, KPAD // 16):
            uv = cand_u[pl.ds(jj * 16, 16)]
            pv = lanes + jj * 16
            take = uv > cu
            cu = jnp.where(take, uv, cu)
            cp = jnp.where(take, pv, cp)
        m = jnp.max(cu)
        posm = jnp.where(cu == m, cp, jnp.int32(100000))
        pos = jnp.min(posm)
        posv = jnp.broadcast_to(pos, (16,))
        plsc.store_scatter(out_pos, [zeros16 + r], posv, mask=lane0)
        plsc.store_scatter(cand_u, [posv], jnp.full((16,), -2, jnp.int32),
                           mask=lane0)
        return 0

    out_pos[pl.ds(0, 16)] = zeros16
    out_pos[pl.ds(KPAD - 16, 16)] = zeros16
    lax.fori_loop(0, K, s_body, 0)

    wv = scale_v[pl.ds(0, 16)]
    hv = scale_v[pl.ds(16, 16)]
    half = jnp.float32(0.5)
    for jj in range(KPAD // 16):
        pv = out_pos[pl.ds(jj * 16, 16)]
        gidx = plsc.load_gather(cand_idx, [pv])
        score = plsc.load_gather(p_v, [gidx])
        q = gidx // C
        lab = gidx - q * C
        q4 = q * 4
        cx = plsc.load_gather(boxes_v, [q4])
        cy = plsc.load_gather(boxes_v, [q4 + 1])
        w = plsc.load_gather(boxes_v, [q4 + 2])
        h = plsc.load_gather(boxes_v, [q4 + 3])
        hw = half * w
        hh = half * h
        scores_v[pl.ds(jj * 16, 16)] = score
        labels_v[pl.ds(jj * 16, 16)] = lab
        boxes_ov[pl.ds(0 * KPAD + jj * 16, 16)] = (cx - hw) * wv
        boxes_ov[pl.ds(1 * KPAD + jj * 16, 16)] = (cy - hh) * hv
        boxes_ov[pl.ds(2 * KPAD + jj * 16, 16)] = (cx + hw) * wv
        boxes_ov[pl.ds(3 * KPAD + jj * 16, 16)] = (cy + hh) * hv

    pltpu.sync_copy(scores_v, scores_hbm.at[wid])
    pltpu.sync_copy(labels_v, labels_hbm.at[wid])
    pltpu.sync_copy(boxes_ov, boxes_out_hbm.at[wid])


def _sc_topk(p, boxes_flat, scale):
    mesh = plsc.VectorSubcoreMesh(core_axis_name="c", subcore_axis_name="s")
    f = pl.kernel(
        _sc_body,
        out_type=(
            jax.ShapeDtypeStruct((B, KPAD), jnp.int32),
            jax.ShapeDtypeStruct((B, KPAD), jnp.int32),
            jax.ShapeDtypeStruct((B, 4 * KPAD), jnp.float32),
        ),
        mesh=mesh,
        compiler_params=pltpu.CompilerParams(needs_layout_passes=False),
        scratch_types=[
            pltpu.VMEM((NPAD,), jnp.int32),
            pltpu.VMEM((4 * Q,), jnp.float32),
            pltpu.VMEM((32,), jnp.float32),
            pltpu.VMEM((HIST,), jnp.int32),
            pltpu.VMEM((KPAD + 16,), jnp.int32),
            pltpu.VMEM((KPAD + 16,), jnp.int32),
            pltpu.VMEM((KPAD,), jnp.int32),
            pltpu.VMEM((KPAD,), jnp.int32),
            pltpu.VMEM((KPAD,), jnp.int32),
            pltpu.VMEM((4 * KPAD,), jnp.float32),
        ],
    )
    return f(p, boxes_flat, scale)


def kernel(pred_logits, pred_boxes, target_sizes):
    x = pred_logits.reshape(B, N)
    xp = jnp.pad(x, ((0, 0), (0, NPAD - N)), constant_values=-1e38)
    p = pl.pallas_call(
        _sig_body,
        out_shape=jax.ShapeDtypeStruct((B, NPAD), jnp.float32),
    )(xp)
    boxes_flat = pred_boxes.reshape(B, 4 * Q)
    ts = target_sizes.astype(jnp.float32)
    img_h = ts[:, 0]
    img_w = ts[:, 1]
    scale = jnp.concatenate(
        [jnp.broadcast_to(img_w[:, None], (B, 16)),
         jnp.broadcast_to(img_h[:, None], (B, 16))], axis=1)
    p_bits = jax.lax.bitcast_convert_type(p, jnp.int32)
    scores_p, labels_p, boxes_p = _sc_topk(p_bits, boxes_flat, scale)
    scores = jax.lax.bitcast_convert_type(scores_p[:, :K], jnp.float32)
    labels = labels_p[:, :K]
    boxes = boxes_p.reshape(B, 4, KPAD).transpose(0, 2, 1)[:, :K, :]
    return (scores, labels, boxes)


# speedup vs baseline: 8.1709x; 1.3474x over previous
import functools

import jax
import jax.numpy as jnp
from jax import lax
from jax.experimental import pallas as pl
from jax.experimental.pallas import tpu as pltpu
from jax.experimental.pallas import tpu_sc as plsc

B, Q, C = 32, 900, 91
N = Q * C
NPAD = 81920
K = 300
KPAD = 304
NVEC = NPAD // 16
UNROLL = 4
HIST = 8192
CAP = 4096


def _sig_body(x_ref, o_ref):
    o_ref[...] = jax.nn.sigmoid(x_ref[...])


def _scan_hist(hist_ref, nbuckets, need):
    nchunks = nbuckets // 16
    lanes = lax.iota(jnp.int32, 16)

    def body(k, carry):
        run, found, b, s_above = carry
        chunk = nchunks - 1 - k
        c = hist_ref[pl.ds(chunk * 16, 16)]
        cr = lax.rev(c, (0,))
        cs = plsc.cumsum(cr)
        s = cs + run
        crossed = s >= need
        n = plsc.all_reduce_ffs(crossed)
        n0 = jnp.min(n)
        hit = jnp.logical_and(jnp.logical_not(found), n0 < 16)
        s_at = jnp.max(jnp.where(lanes == n0, s, 0))
        cr_at = jnp.max(jnp.where(lanes == n0, cr, 0))
        b_new = jnp.where(hit, chunk * 16 + 15 - n0, b)
        s_above_new = jnp.where(hit, s_at - cr_at, s_above)
        found_new = jnp.logical_or(found, n0 < 16)
        run_new = run + jnp.max(cs)
        return run_new, found_new, b_new, s_above_new

    init = (jnp.int32(0), jnp.bool_(False), jnp.int32(0), jnp.int32(0))
    _, _, b, s_above = lax.fori_loop(0, nchunks, body, init)
    return b, s_above


def _zero(ref, nwords):
    zeros = jnp.zeros((16,), jnp.int32)

    def body(k, _):
        ref[pl.ds(k * 16, 16)] = zeros
        return 0

    lax.fori_loop(0, nwords // 16, body, 0)


def _sc_body(p_hbm, boxes_hbm, scale_hbm,
             scores_hbm, labels_hbm, boxes_out_hbm,
             p_v, boxes_v, scale_v, hist_v,
             coarse_u, coarse_idx,
             cand_u, cand_idx, out_pos,
             scores_v, labels_v, boxes_ov):
    nc = 2
    wid = lax.axis_index("s") * nc + lax.axis_index("c")
    lanes = lax.iota(jnp.int32, 16)
    ones16 = jnp.ones((16,), jnp.int32)

    pltpu.sync_copy(p_hbm.at[wid], p_v)
    pltpu.sync_copy(boxes_hbm.at[wid], boxes_v)
    pltpu.sync_copy(scale_hbm.at[wid], scale_v)

    _zero(hist_v, HIST)

    def r0_body(j, _):
        for k in range(UNROLL):
            u = p_v[pl.ds((j * UNROLL + k) * 16, 16)]
            b = jnp.right_shift(u, 17)
            plsc.addupdate_scatter(hist_v, [b], ones16)
        return 0

    lax.fori_loop(0, NVEC // UNROLL, r0_body, 0)
    b0, gt0 = _scan_hist(hist_v, HIST, K)

    t0 = b0 << 17
    neg1 = jnp.full((16,), -1, jnp.int32)

    def cc_body(j, cnt):
        for k in range(UNROLL):
            base = (j * UNROLL + k) * 16
            u = p_v[pl.ds(base, 16)]
            m = u >= t0
            off = jnp.minimum(cnt, CAP)
            plsc.store_compressed(coarse_u.at[pl.ds(off, 16)], u, mask=m)
            plsc.store_compressed(coarse_idx.at[pl.ds(off, 16)], lanes + base,
                                  mask=m)
            cnt = cnt + jnp.min(plsc.all_reduce_population_count(m))
        return cnt

    ccnt = lax.fori_loop(0, NVEC // UNROLL, cc_body, jnp.int32(0))
    ccnt_c = jnp.minimum(ccnt, jnp.int32(CAP))
    coarse_u[pl.ds(ccnt_c, 16)] = neg1
    nvc = jnp.right_shift(ccnt_c + 15, 4)

    _zero(hist_v, 512)
    need1 = K - gt0

    def r1_body(j, _):
        u = coarse_u[pl.ds(j * 16, 16)]
        pm = jnp.right_shift(u, 17) == b0
        d = jnp.right_shift(u, 8) & 0x1FF
        plsc.addupdate_scatter(hist_v, [d], ones16, mask=pm)
        return 0

    lax.fori_loop(0, nvc, r1_body, 0)
    b1, gt1 = _scan_hist(hist_v, 512, need1)
    p2 = (b0 << 9) | b1

    _zero(hist_v, 256)
    need2 = need1 - gt1

    def r2_body(j, _):
        u = coarse_u[pl.ds(j * 16, 16)]
        pm = jnp.right_shift(u, 8) == p2
        d = u & 0xFF
        plsc.addupdate_scatter(hist_v, [d], ones16, mask=pm)
        return 0

    lax.fori_loop(0, nvc, r2_body, 0)
    b2, gt2 = _scan_hist(hist_v, 256, need2)
    t = (p2 << 8) | b2
    cnt_gt = gt0 + gt1 + gt2
    need_eq = K - cnt_gt

    for j in range((KPAD + 16) // 16):
        cand_u[pl.ds(j * 16, 16)] = neg1

    def c_body(j, carry):
        sel_cnt, eq_cnt = carry
        u = coarse_u[pl.ds(j * 16, 16)]
        idxv = coarse_idx[pl.ds(j * 16, 16)]
        msel = u > t
        meq = u == t
        plsc.store_compressed(cand_u.at[pl.ds(sel_cnt, 16)], u, mask=msel)
        plsc.store_compressed(cand_idx.at[pl.ds(sel_cnt, 16)], idxv,
                              mask=msel)
        nsel = jnp.min(plsc.all_reduce_population_count(msel))
        sel_cnt = sel_cnt + nsel

        eq_open = eq_cnt < need_eq

        @pl.when(eq_open)
        def _():
            off = cnt_gt + eq_cnt
            plsc.store_compressed(cand_u.at[pl.ds(off, 16)], u, mask=meq)
            plsc.store_compressed(cand_idx.at[pl.ds(off, 16)], idxv,
                                  mask=meq)

        neq = jnp.min(plsc.all_reduce_population_count(meq))
        eq_cnt = jnp.where(eq_open, eq_cnt + neq, eq_cnt)
        return sel_cnt, eq_cnt

    lax.fori_loop(0, nvc, c_body, (jnp.int32(0), jnp.int32(0)))

    lane0 = lanes == 0
    zeros16 = jnp.zeros((16,), jnp.int32)

    def s_body(r, _):
        cu = cand_u[pl.ds(0, 16)]
        cp = lanes
        for jj in range(1, KPAD // 16):
            uv = cand_u[pl.ds(jj * 16, 16)]
            pv = lanes + jj * 16
            take = uv > cu
            cu = jnp.where(take, uv, cu)
            cp = jnp.where(take, pv, cp)
        m = jnp.max(cu)
        posm = jnp.where(cu == m, cp, jnp.int32(100000))
        pos = jnp.min(posm)
        posv = jnp.broadcast_to(pos, (16,))
        plsc.store_scatter(out_pos, [zeros16 + r], posv, mask=lane0)
        plsc.store_scatter(cand_u, [posv], jnp.full((16,), -2, jnp.int32),
                           mask=lane0)
        return 0

    out_pos[pl.ds(0, 16)] = zeros16
    out_pos[pl.ds(KPAD - 16, 16)] = zeros16
    lax.fori_loop(0, K, s_body, 0)

    wv = scale_v[pl.ds(0, 16)]
    hv = scale_v[pl.ds(16, 16)]
    half = jnp.float32(0.5)
    for jj in range(KPAD // 16):
        pv = out_pos[pl.ds(jj * 16, 16)]
        gidx = plsc.load_gather(cand_idx, [pv])
        score = plsc.load_gather(p_v, [gidx])
        q = gidx // C
        lab = gidx - q * C
        q4 = q * 4
        cx = plsc.load_gather(boxes_v, [q4])
        cy = plsc.load_gather(boxes_v, [q4 + 1])
        w = plsc.load_gather(boxes_v, [q4 + 2])
        h = plsc.load_gather(boxes_v, [q4 + 3])
        hw = half * w
        hh = half * h
        scores_v[pl.ds(jj * 16, 16)] = score
        labels_v[pl.ds(jj * 16, 16)] = lab
        boxes_ov[pl.ds(0 * KPAD + jj * 16, 16)] = (cx - hw) * wv
        boxes_ov[pl.ds(1 * KPAD + jj * 16, 16)] = (cy - hh) * hv
        boxes_ov[pl.ds(2 * KPAD + jj * 16, 16)] = (cx + hw) * wv
        boxes_ov[pl.ds(3 * KPAD + jj * 16, 16)] = (cy + hh) * hv

    pltpu.sync_copy(scores_v, scores_hbm.at[wid])
    pltpu.sync_copy(labels_v, labels_hbm.at[wid])
    pltpu.sync_copy(boxes_ov, boxes_out_hbm.at[wid])


def _sc_topk(p, boxes_flat, scale):
    mesh = plsc.VectorSubcoreMesh(core_axis_name="c", subcore_axis_name="s")
    f = pl.kernel(
        _sc_body,
        out_type=(
            jax.ShapeDtypeStruct((B, KPAD), jnp.int32),
            jax.ShapeDtypeStruct((B, KPAD), jnp.int32),
            jax.ShapeDtypeStruct((B, 4 * KPAD), jnp.float32),
        ),
        mesh=mesh,
        compiler_params=pltpu.CompilerParams(needs_layout_passes=False),
        scratch_types=[
            pltpu.VMEM((NPAD,), jnp.int32),
            pltpu.VMEM((4 * Q,), jnp.float32),
            pltpu.VMEM((32,), jnp.float32),
            pltpu.VMEM((HIST,), jnp.int32),
            pltpu.VMEM((CAP + 32,), jnp.int32),
            pltpu.VMEM((CAP + 32,), jnp.int32),
            pltpu.VMEM((KPAD + 16,), jnp.int32),
            pltpu.VMEM((KPAD + 16,), jnp.int32),
            pltpu.VMEM((KPAD,), jnp.int32),
            pltpu.VMEM((KPAD,), jnp.int32),
            pltpu.VMEM((KPAD,), jnp.int32),
            pltpu.VMEM((4 * KPAD,), jnp.float32),
        ],
    )
    return f(p, boxes_flat, scale)


def kernel(pred_logits, pred_boxes, target_sizes):
    x = pred_logits.reshape(B, N)
    xp = jnp.pad(x, ((0, 0), (0, NPAD - N)), constant_values=-1e38)
    p = pl.pallas_call(
        _sig_body,
        out_shape=jax.ShapeDtypeStruct((B, NPAD), jnp.float32),
    )(xp)
    boxes_flat = pred_boxes.reshape(B, 4 * Q)
    ts = target_sizes.astype(jnp.float32)
    img_h = ts[:, 0]
    img_w = ts[:, 1]
    scale = jnp.concatenate(
        [jnp.broadcast_to(img_w[:, None], (B, 16)),
         jnp.broadcast_to(img_h[:, None], (B, 16))], axis=1)
    p_bits = jax.lax.bitcast_convert_type(p, jnp.int32)
    scores_p, labels_p, boxes_p = _sc_topk(p_bits, boxes_flat, scale)
    scores = jax.lax.bitcast_convert_type(scores_p[:, :K], jnp.float32)
    labels = labels_p[:, :K]
    boxes = boxes_p.reshape(B, 4, KPAD).transpose(0, 2, 1)[:, :K, :]
    return (scores, labels, boxes)


# speedup vs baseline: 8.1738x; 1.0004x over previous
import functools

import jax
import jax.numpy as jnp
from jax import lax
from jax.experimental import pallas as pl
from jax.experimental.pallas import tpu as pltpu
from jax.experimental.pallas import tpu_sc as plsc

B, Q, C = 32, 900, 91
N = Q * C
NPAD = 81920
K = 300
KPAD = 304
NVEC = NPAD // 16
UNROLL = 4
HIST = 8192
CAP = 4096


def _sig_body(x_ref, o_ref):
    o_ref[...] = jax.nn.sigmoid(x_ref[...])


def _scan_hist(hist_ref, nbuckets, need):
    nchunks = nbuckets // 16
    lanes = lax.iota(jnp.int32, 16)

    def body(k, carry):
        run, found, b, s_above = carry
        chunk = nchunks - 1 - k
        c = hist_ref[pl.ds(chunk * 16, 16)]
        cr = lax.rev(c, (0,))
        cs = plsc.cumsum(cr)
        s = cs + run
        crossed = s >= need
        n = plsc.all_reduce_ffs(crossed)
        n0 = jnp.min(n)
        hit = jnp.logical_and(jnp.logical_not(found), n0 < 16)
        s_at = jnp.max(jnp.where(lanes == n0, s, 0))
        cr_at = jnp.max(jnp.where(lanes == n0, cr, 0))
        b_new = jnp.where(hit, chunk * 16 + 15 - n0, b)
        s_above_new = jnp.where(hit, s_at - cr_at, s_above)
        found_new = jnp.logical_or(found, n0 < 16)
        run_new = run + jnp.max(cs)
        return run_new, found_new, b_new, s_above_new

    init = (jnp.int32(0), jnp.bool_(False), jnp.int32(0), jnp.int32(0))
    _, _, b, s_above = lax.fori_loop(0, nchunks, body, init)
    return b, s_above


def _zero(ref, nwords):
    zeros = jnp.zeros((16,), jnp.int32)

    def body(k, _):
        ref[pl.ds(k * 16, 16)] = zeros
        return 0

    lax.fori_loop(0, nwords // 16, body, 0)


def _sc_body(p_hbm, boxes_hbm, scale_hbm,
             scores_hbm, labels_hbm, boxes_out_hbm,
             p_v, boxes_v, scale_v, hist_v,
             coarse_u, coarse_idx,
             cand_u, cand_idx, out_pos,
             scores_v, labels_v, boxes_ov):
    nc = 2
    wid = lax.axis_index("s") * nc + lax.axis_index("c")
    lanes = lax.iota(jnp.int32, 16)
    ones16 = jnp.ones((16,), jnp.int32)

    pltpu.sync_copy(p_hbm.at[wid], p_v)
    pltpu.sync_copy(boxes_hbm.at[wid], boxes_v)
    pltpu.sync_copy(scale_hbm.at[wid], scale_v)

    _zero(hist_v, HIST)

    def r0_body(j, _):
        for k in range(UNROLL):
            u = p_v[pl.ds((j * UNROLL + k) * 16, 16)]
            b = jnp.right_shift(u, 17)
            plsc.addupdate_scatter(hist_v, [b], ones16)
        return 0

    lax.fori_loop(0, NVEC // UNROLL, r0_body, 0)
    b0, gt0 = _scan_hist(hist_v, HIST, K)

    t0 = b0 << 17
    neg1 = jnp.full((16,), -1, jnp.int32)

    def cc_body(j, cnt):
        for k in range(UNROLL):
            base = (j * UNROLL + k) * 16
            u = p_v[pl.ds(base, 16)]
            m = u >= t0
            off = jnp.minimum(cnt, CAP)
            plsc.store_compressed(coarse_u.at[pl.ds(off, 16)], u, mask=m)
            plsc.store_compressed(coarse_idx.at[pl.ds(off, 16)], lanes + base,
                                  mask=m)
            cnt = cnt + jnp.min(plsc.all_reduce_population_count(m))
        return cnt

    ccnt = lax.fori_loop(0, NVEC // UNROLL, cc_body, jnp.int32(0))
    ccnt_c = jnp.minimum(ccnt, jnp.int32(CAP))
    coarse_u[pl.ds(ccnt_c, 16)] = neg1
    nvc = jnp.right_shift(ccnt_c + 15, 4)

    _zero(hist_v, 512)
    need1 = K - gt0

    def r1_body(j, _):
        u = coarse_u[pl.ds(j * 16, 16)]
        pm = jnp.right_shift(u, 17) == b0
        d = jnp.right_shift(u, 8) & 0x1FF
        plsc.addupdate_scatter(hist_v, [d], ones16, mask=pm)
        return 0

    lax.fori_loop(0, nvc, r1_body, 0)
    b1, gt1 = _scan_hist(hist_v, 512, need1)
    p2 = (b0 << 9) | b1

    _zero(hist_v, 256)
    need2 = need1 - gt1

    def r2_body(j, _):
        u = coarse_u[pl.ds(j * 16, 16)]
        pm = jnp.right_shift(u, 8) == p2
        d = u & 0xFF
        plsc.addupdate_scatter(hist_v, [d], ones16, mask=pm)
        return 0

    lax.fori_loop(0, nvc, r2_body, 0)
    b2, gt2 = _scan_hist(hist_v, 256, need2)
    t = (p2 << 8) | b2
    cnt_gt = gt0 + gt1 + gt2
    need_eq = K - cnt_gt

    for j in range((KPAD + 16) // 16):
        cand_u[pl.ds(j * 16, 16)] = neg1

    def c_body(j, carry):
        sel_cnt, eq_cnt = carry
        u = coarse_u[pl.ds(j * 16, 16)]
        idxv = coarse_idx[pl.ds(j * 16, 16)]
        msel = u > t
        meq = u == t
        plsc.store_compressed(cand_u.at[pl.ds(sel_cnt, 16)], u, mask=msel)
        plsc.store_compressed(cand_idx.at[pl.ds(sel_cnt, 16)], idxv,
                              mask=msel)
        nsel = jnp.min(plsc.all_reduce_population_count(msel))
        sel_cnt = sel_cnt + nsel

        eq_open = eq_cnt < need_eq

        @pl.when(eq_open)
        def _():
            off = cnt_gt + eq_cnt
            plsc.store_compressed(cand_u.at[pl.ds(off, 16)], u, mask=meq)
            plsc.store_compressed(cand_idx.at[pl.ds(off, 16)], idxv,
                                  mask=meq)

        neq = jnp.min(plsc.all_reduce_population_count(meq))
        eq_cnt = jnp.where(eq_open, eq_cnt + neq, eq_cnt)
        return sel_cnt, eq_cnt

    lax.fori_loop(0, nvc, c_body, (jnp.int32(0), jnp.int32(0)))

    lane0 = lanes == 0
    zeros16 = jnp.zeros((16,), jnp.int32)

    def s_body(r, _):
        cu = cand_u[pl.ds(0, 16)]
        cp = lanes
        for jj in range(1, KPAD // 16):
            uv = cand_u[pl.ds(jj * 16, 16)]
            pv = lanes + jj * 16
            take = uv > cu
            cu = jnp.where(take, uv, cu)
            cp = jnp.where(take, pv, cp)
        m = jnp.max(cu)
        posm = jnp.where(cu == m, cp, jnp.int32(100000))
        pos = jnp.min(posm)
        posv = jnp.broadcast_to(pos, (16,))
        plsc.store_scatter(out_pos, [zeros16 + r], posv, mask=lane0)
        plsc.store_scatter(cand_u, [posv], jnp.full((16,), -2, jnp.int32),
                           mask=lane0)
        return 0

    out_pos[pl.ds(0, 16)] = zeros16
    out_pos[pl.ds(KPAD - 16, 16)] = zeros16
    lax.fori_loop(0, K, s_body, 0)

    wv = scale_v[pl.ds(0, 16)]
    hv = scale_v[pl.ds(16, 16)]
    half = jnp.float32(0.5)
    for jj in range(KPAD // 16):
        pv = out_pos[pl.ds(jj * 16, 16)]
        gidx = plsc.load_gather(cand_idx, [pv])
        score = plsc.load_gather(p_v, [gidx])
        q = gidx // C
        lab = gidx - q * C
        q4 = q * 4
        cx = plsc.load_gather(boxes_v, [q4])
        cy = plsc.load_gather(boxes_v, [q4 + 1])
        w = plsc.load_gather(boxes_v, [q4 + 2])
        h = plsc.load_gather(boxes_v, [q4 + 3])
        hw = half * w
        hh = half * h
        scores_v[pl.ds(jj * 16, 16)] = score
        labels_v[pl.ds(jj * 16, 16)] = lab
        boxes_ov[pl.ds(0 * KPAD + jj * 16, 16)] = (cx - hw) * wv
        boxes_ov[pl.ds(1 * KPAD + jj * 16, 16)] = (cy - hh) * hv
        boxes_ov[pl.ds(2 * KPAD + jj * 16, 16)] = (cx + hw) * wv
        boxes_ov[pl.ds(3 * KPAD + jj * 16, 16)] = (cy + hh) * hv

    pltpu.sync_copy(scores_v, scores_hbm.at[wid])
    pltpu.sync_copy(labels_v, labels_hbm.at[wid])
    pltpu.sync_copy(boxes_ov, boxes_out_hbm.at[wid])


def _sc_topk(p, boxes_flat, scale):
    mesh = plsc.VectorSubcoreMesh(core_axis_name="c", subcore_axis_name="s")
    f = pl.kernel(
        _sc_body,
        out_type=(
            jax.ShapeDtypeStruct((B, KPAD), jnp.int32),
            jax.ShapeDtypeStruct((B, KPAD), jnp.int32),
            jax.ShapeDtypeStruct((B, 4 * KPAD), jnp.float32),
        ),
        mesh=mesh,
        compiler_params=pltpu.CompilerParams(needs_layout_passes=False, use_tc_tiling_on_sc=True),
        scratch_types=[
            pltpu.VMEM((NPAD,), jnp.int32),
            pltpu.VMEM((4 * Q,), jnp.float32),
            pltpu.VMEM((32,), jnp.float32),
            pltpu.VMEM((HIST,), jnp.int32),
            pltpu.VMEM((CAP + 32,), jnp.int32),
            pltpu.VMEM((CAP + 32,), jnp.int32),
            pltpu.VMEM((KPAD + 16,), jnp.int32),
            pltpu.VMEM((KPAD + 16,), jnp.int32),
            pltpu.VMEM((KPAD,), jnp.int32),
            pltpu.VMEM((KPAD,), jnp.int32),
            pltpu.VMEM((KPAD,), jnp.int32),
            pltpu.VMEM((4 * KPAD,), jnp.float32),
        ],
    )
    return f(p, boxes_flat, scale)


def kernel(pred_logits, pred_boxes, target_sizes):
    x = pred_logits.reshape(B, N)
    xp = jnp.pad(x, ((0, 0), (0, NPAD - N)), constant_values=-1e38)
    p = pl.pallas_call(
        _sig_body,
        out_shape=jax.ShapeDtypeStruct((B, NPAD), jnp.float32),
    )(xp)
    boxes_flat = pred_boxes.reshape(B, 4 * Q)
    ts = target_sizes.astype(jnp.float32)
    img_h = ts[:, 0]
    img_w = ts[:, 1]
    scale = jnp.concatenate(
        [jnp.broadcast_to(img_w[:, None], (B, 16)),
         jnp.broadcast_to(img_h[:, None], (B, 16))], axis=1)
    p_bits = jax.lax.bitcast_convert_type(p, jnp.int32)
    scores_p, labels_p, boxes_p = _sc_topk(p_bits, boxes_flat, scale)
    scores = jax.lax.bitcast_convert_type(scores_p[:, :K], jnp.float32)
    labels = labels_p[:, :K]
    boxes = boxes_p.reshape(B, 4, KPAD).transpose(0, 2, 1)[:, :K, :]
    return (scores, labels, boxes)


# speedup vs baseline: 11.7621x; 1.4390x over previous
import jax
import jax.numpy as jnp
from jax import lax
from jax.experimental import pallas as pl
from jax.experimental.pallas import tpu as pltpu
from jax.experimental.pallas import tpu_sc as plsc

B, Q, C = 32, 900, 91
QP = 912
NP2 = C * QP
NPV = NP2 + 16
NVEC = NPV // 16
K = 300
KPAD = 304
UNROLL = 4
HIST = 8192
CAP = 4096
IDXPAD = (1 << 17) - 1


def _sig_body(x_ref, o_ref):
    p = jax.nn.sigmoid(x_ref[...])
    o_ref[:, :, :Q] = lax.bitcast_convert_type(p, jnp.int32)
    o_ref[:, :, Q:] = jnp.full((C, B, QP - Q), -1, jnp.int32)


def _scan_hist(hist_ref, nbuckets, need):
    nchunks = nbuckets // 16
    lanes = lax.iota(jnp.int32, 16)

    def cond(st):
        c, run, t = st
        return run + t < need

    def wbody(st):
        c, run, t = st
        c2 = c - 1
        return c2, run + t, jnp.sum(hist_ref[pl.ds(c2 * 16, 16)])

    c0 = jnp.int32(nchunks - 1)
    st = (c0, jnp.int32(0), jnp.sum(hist_ref[pl.ds(c0 * 16, 16)]))
    c, run, _ = lax.while_loop(cond, wbody, st)
    cvec = hist_ref[pl.ds(c * 16, 16)]
    cr = lax.rev(cvec, (0,))
    cs = plsc.cumsum(cr)
    s = cs + run
    n = plsc.all_reduce_ffs(s >= need)
    n0 = jnp.min(n)
    s_at = jnp.max(jnp.where(lanes == n0, s, 0))
    cr_at = jnp.max(jnp.where(lanes == n0, cr, 0))
    return c * 16 + 15 - n0, s_at - cr_at


def _sc_body(p_hbm, boxes_hbm, scale_hbm,
             scores_hbm, labels_hbm, boxes_out_hbm,
             p_v, boxes_v, scale_v, hist_v,
             coarse_u, coarse_idx,
             cand_u, cand_idx, cand2_u, cand2_idx,
             scores_v, labels_v, boxes_ov, sem):
    nc = 2
    wid = lax.axis_index("s") * nc + lax.axis_index("c")
    lanes = lax.iota(jnp.int32, 16)
    ones16 = jnp.ones((16,), jnp.int32)
    zeros16 = jnp.zeros((16,), jnp.int32)
    neg1 = jnp.full((16,), -1, jnp.int32)

    base_in = wid * QP
    copies = [
        pltpu.async_copy(p_hbm.at[pl.ds(c * (B * QP) + base_in, QP)],
                         p_v.at[pl.ds(c * QP, QP)], sem)
        for c in range(C)
    ]
    copies.append(pltpu.async_copy(boxes_hbm.at[wid], boxes_v, sem))
    copies.append(pltpu.async_copy(scale_hbm.at[wid], scale_v, sem))
    p_v[pl.ds(NP2, 16)] = neg1

    def z_body(k, _):
        hist_v[pl.ds(k * 16, 16)] = zeros16
        return 0

    lax.fori_loop(0, HIST // 16, z_body, 0)
    for cp in copies:
        cp.wait()

    def r0_body(j, _):
        for k in range(UNROLL):
            u = p_v[pl.ds((j * UNROLL + k) * 16, 16)]
            b = jnp.right_shift(u, 17) & 0x1FFF
            plsc.addupdate_scatter(hist_v, [b], ones16, mask=u >= 0)
        return 0

    lax.fori_loop(0, NVEC // UNROLL, r0_body, 0)
    b0, gt0 = _scan_hist(hist_v, HIST, K)

    t0 = b0 << 17

    def cc_body(j, cnt):
        for k in range(UNROLL):
            base = (j * UNROLL + k) * 16
            u = p_v[pl.ds(base, 16)]
            m = u >= t0
            off = jnp.minimum(cnt, CAP)
            plsc.store_compressed(coarse_u.at[pl.ds(off, 16)], u, mask=m)
            plsc.store_compressed(coarse_idx.at[pl.ds(off, 16)], lanes + base,
                                  mask=m)
            cnt = cnt + jnp.min(plsc.all_reduce_population_count(m))
        return cnt

    ccnt = lax.fori_loop(0, NVEC // UNROLL, cc_body, jnp.int32(0))
    ccnt_c = jnp.minimum(ccnt, jnp.int32(CAP))
    coarse_u[pl.ds(ccnt_c, 16)] = neg1
    nvc = jnp.right_shift(ccnt_c + 15, 4)

    for i in range(32):
        hist_v[pl.ds(i * 16, 16)] = zeros16
    need1 = K - gt0

    def r1_body(j, _):
        u = coarse_u[pl.ds(j * 16, 16)]
        pm = jnp.right_shift(u, 17) == b0
        d = jnp.right_shift(u, 8) & 0x1FF
        plsc.addupdate_scatter(hist_v, [d], ones16, mask=pm)
        return 0

    lax.fori_loop(0, nvc, r1_body, 0)
    b1, gt1 = _scan_hist(hist_v, 512, need1)
    p2 = (b0 << 9) | b1

    for i in range(16):
        hist_v[pl.ds(i * 16, 16)] = zeros16
    need2 = need1 - gt1

    def r2_body(j, _):
        u = coarse_u[pl.ds(j * 16, 16)]
        pm = jnp.right_shift(u, 8) == p2
        d = u & 0xFF
        plsc.addupdate_scatter(hist_v, [d], ones16, mask=pm)
        return 0

    lax.fori_loop(0, nvc, r2_body, 0)
    b2, gt2 = _scan_hist(hist_v, 256, need2)
    t = (p2 << 8) | b2
    cnt_gt = gt0 + gt1 + gt2

    idxpad16 = jnp.full((16,), IDXPAD, jnp.int32)
    for j in range((KPAD + 16) // 16):
        cand_u[pl.ds(j * 16, 16)] = zeros16
        cand_idx[pl.ds(j * 16, 16)] = idxpad16

    def c_body(j, carry):
        sel_cnt, eq_cnt = carry
        u = coarse_u[pl.ds(j * 16, 16)]
        raw = coarse_idx[pl.ds(j * 16, 16)]
        cc = raw // QP
        ridx = (raw - cc * QP) * C + cc
        msel = u > t
        meq = u == t
        off1 = jnp.minimum(sel_cnt, jnp.int32(KPAD))
        plsc.store_compressed(cand_u.at[pl.ds(off1, 16)], u, mask=msel)
        plsc.store_compressed(cand_idx.at[pl.ds(off1, 16)], ridx, mask=msel)
        sel_cnt = sel_cnt + jnp.min(plsc.all_reduce_population_count(msel))
        off2 = jnp.minimum(cnt_gt + eq_cnt, jnp.int32(KPAD))
        plsc.store_compressed(cand_u.at[pl.ds(off2, 16)], u, mask=meq)
        plsc.store_compressed(cand_idx.at[pl.ds(off2, 16)], ridx, mask=meq)
        eq_cnt = eq_cnt + jnp.min(plsc.all_reduce_population_count(meq))
        return sel_cnt, eq_cnt

    lax.fori_loop(0, nvc, c_body, (jnp.int32(0), jnp.int32(0)))

    NCAND = (KPAD + 16) // 16

    def radix_pass(digit_fn, nbuckets, su, si, du, di):
        for i in range(nbuckets // 16):
            hist_v[pl.ds(i * 16, 16)] = zeros16
        for j in range(NCAND):
            d = digit_fn(su[pl.ds(j * 16, 16)], si[pl.ds(j * 16, 16)])
            cnt, last = plsc.scan_count(d)
            plsc.addupdate_scatter(hist_v, [d], cnt, mask=last)
        run = jnp.int32(0)
        for i in range(nbuckets // 16):
            cvec = hist_v[pl.ds(i * 16, 16)]
            cs = plsc.cumsum(cvec)
            hist_v[pl.ds(i * 16, 16)] = cs - cvec + run
            run = run + jnp.max(cs)
        for j in range(NCAND):
            u = su[pl.ds(j * 16, 16)]
            iv = si[pl.ds(j * 16, 16)]
            d = digit_fn(u, iv)
            cnt, last = plsc.scan_count(d)
            base = plsc.load_gather(hist_v, [d])
            pos = base + cnt - 1
            plsc.store_scatter(du, [pos], u)
            plsc.store_scatter(di, [pos], iv)
            plsc.addupdate_scatter(hist_v, [d], cnt, mask=last)

    a = (cand_u, cand_idx)
    bb = (cand2_u, cand2_idx)
    passes = [
        (lambda u, i: i & 0x1FF, 512),
        (lambda u, i: jnp.right_shift(i, 9) & 0xFF, 256),
        (lambda u, i: (~u) & 0xFF, 256),
        (lambda u, i: jnp.right_shift(~u, 8) & 0xFF, 256),
        (lambda u, i: jnp.right_shift(~u, 16) & 0xFF, 256),
        (lambda u, i: jnp.right_shift(~u, 24) & 0xFF, 256),
    ]
    for fn, nb in passes:
        radix_pass(fn, nb, a[0], a[1], bb[0], bb[1])
        a, bb = bb, a

    wv = scale_v[pl.ds(0, 16)]
    hv = scale_v[pl.ds(16, 16)]
    half = jnp.float32(0.5)
    for jj in range(KPAD // 16):
        score = cand_u[pl.ds(jj * 16, 16)]
        ridx = cand_idx[pl.ds(jj * 16, 16)]
        qfull = ridx // C
        q = jnp.minimum(qfull, Q - 1)
        lab = ridx - qfull * C
        cx = plsc.load_gather(boxes_v, [q])
        cy = plsc.load_gather(boxes_v, [q + Q])
        w = plsc.load_gather(boxes_v, [q + 2 * Q])
        h = plsc.load_gather(boxes_v, [q + 3 * Q])
        hw = half * w
        hh = half * h
        scores_v[pl.ds(jj * 16, 16)] = score
        labels_v[pl.ds(jj * 16, 16)] = lab
        boxes_ov[pl.ds(0 * KPAD + jj * 16, 16)] = (cx - hw) * wv
        boxes_ov[pl.ds(1 * KPAD + jj * 16, 16)] = (cy - hh) * hv
        boxes_ov[pl.ds(2 * KPAD + jj * 16, 16)] = (cx + hw) * wv
        boxes_ov[pl.ds(3 * KPAD + jj * 16, 16)] = (cy + hh) * hv

    pltpu.sync_copy(scores_v, scores_hbm.at[wid])
    pltpu.sync_copy(labels_v, labels_hbm.at[wid])
    pltpu.sync_copy(boxes_ov, boxes_out_hbm.at[wid])


def _sc_topk(p, boxes_flat, scale):
    mesh = plsc.VectorSubcoreMesh(core_axis_name="c", subcore_axis_name="s")
    f = pl.kernel(
        _sc_body,
        out_type=(
            jax.ShapeDtypeStruct((B, KPAD), jnp.int32),
            jax.ShapeDtypeStruct((B, KPAD), jnp.int32),
            jax.ShapeDtypeStruct((B, 4 * KPAD), jnp.float32),
        ),
        mesh=mesh,
        compiler_params=pltpu.CompilerParams(needs_layout_passes=False),
        scratch_types=[
            pltpu.VMEM((NPV,), jnp.int32),
            pltpu.VMEM((4 * Q,), jnp.float32),
            pltpu.VMEM((32,), jnp.float32),
            pltpu.VMEM((HIST,), jnp.int32),
            pltpu.VMEM((CAP + 32,), jnp.int32),
            pltpu.VMEM((CAP + 32,), jnp.int32),
            pltpu.VMEM((KPAD + 16,), jnp.int32),
            pltpu.VMEM((KPAD + 16,), jnp.int32),
            pltpu.VMEM((KPAD + 16,), jnp.int32),
            pltpu.VMEM((KPAD + 16,), jnp.int32),
            pltpu.VMEM((KPAD,), jnp.int32),
            pltpu.VMEM((KPAD,), jnp.int32),
            pltpu.VMEM((4 * KPAD,), jnp.float32),
            pltpu.SemaphoreType.DMA,
        ],
    )
    return f(p, boxes_flat, scale)


def kernel(pred_logits, pred_boxes, target_sizes):
    x_t = jnp.transpose(pred_logits, (2, 0, 1))
    p = pl.pallas_call(
        _sig_body,
        out_shape=jax.ShapeDtypeStruct((C, B, QP), jnp.int32),
    )(x_t)
    p_flat = p.reshape(C * B * QP)
    boxes_cm = jnp.transpose(pred_boxes, (0, 2, 1)).reshape(B, 4 * Q)
    ts = target_sizes.astype(jnp.float32)
    img_h = ts[:, 0]
    img_w = ts[:, 1]
    scale = jnp.concatenate(
        [jnp.broadcast_to(img_w[:, None], (B, 16)),
         jnp.broadcast_to(img_h[:, None], (B, 16))], axis=1)
    scores_p, labels_p, boxes_p = _sc_topk(p_flat, boxes_cm, scale)
    scores = jax.lax.bitcast_convert_type(scores_p[:, :K], jnp.float32)
    labels = labels_p[:, :K]
    boxes = boxes_p.reshape(B, 4, KPAD).transpose(0, 2, 1)[:, :K, :]
    return (scores, labels, boxes)


# speedup vs baseline: 18.0035x; 1.5306x over previous
import jax
import jax.numpy as jnp
from jax import lax
from jax.experimental import pallas as pl
from jax.experimental.pallas import tpu as pltpu
from jax.experimental.pallas import tpu_sc as plsc

B, Q, C = 32, 900, 91
QP = 912
NP2 = C * QP
NPV = NP2 + 16
NVEC = NPV // 16
K = 300
KPAD = 304
UNROLL = 4
HIST = 8192
CAP = 4096
IDXPAD = (1 << 17) - 1


def _sig_body(x_ref, o_ref):
    p = jax.nn.sigmoid(x_ref[...])
    o_ref[:, :, :Q] = lax.bitcast_convert_type(p, jnp.int32)
    o_ref[:, :, Q:] = jnp.full((C, B, QP - Q), -1, jnp.int32)


def _scan_hist(hist_ref, nbuckets, need):
    nchunks = nbuckets // 16
    lanes = lax.iota(jnp.int32, 16)

    def cond(st):
        c, run, t = st
        return run + t < need

    def wbody(st):
        c, run, t = st
        c2 = c - 1
        return c2, run + t, jnp.sum(hist_ref[pl.ds(c2 * 16, 16)])

    c0 = jnp.int32(nchunks - 1)
    st = (c0, jnp.int32(0), jnp.sum(hist_ref[pl.ds(c0 * 16, 16)]))
    c, run, _ = lax.while_loop(cond, wbody, st)
    cvec = hist_ref[pl.ds(c * 16, 16)]
    cr = lax.rev(cvec, (0,))
    cs = plsc.cumsum(cr)
    s = cs + run
    n = plsc.all_reduce_ffs(s >= need)
    n0 = jnp.min(n)
    s_at = jnp.max(jnp.where(lanes == n0, s, 0))
    cr_at = jnp.max(jnp.where(lanes == n0, cr, 0))
    return c * 16 + 15 - n0, s_at - cr_at


def _sc_body(p_hbm, boxes_hbm, scale_hbm,
             scores_hbm, labels_hbm, boxes_out_hbm,
             p_v, boxes_v, scale_v, hist_v,
             coarse_u, coarse_idx,
             cand_u, cand_idx, cand2_u, cand2_idx,
             scores_v, labels_v, boxes_ov, sem):
    nc = 2
    wid = lax.axis_index("s") * nc + lax.axis_index("c")
    lanes = lax.iota(jnp.int32, 16)
    ones16 = jnp.ones((16,), jnp.int32)
    zeros16 = jnp.zeros((16,), jnp.int32)
    neg1 = jnp.full((16,), -1, jnp.int32)

    base_in = wid * QP
    copies = [
        pltpu.async_copy(p_hbm.at[pl.ds(c * (B * QP) + base_in, QP)],
                         p_v.at[pl.ds(c * QP, QP)], sem)
        for c in range(C)
    ]
    copies.append(pltpu.async_copy(boxes_hbm.at[wid], boxes_v, sem))
    copies.append(pltpu.async_copy(scale_hbm.at[wid], scale_v, sem))
    p_v[pl.ds(NP2, 16)] = neg1

    def z_body(k, _):
        hist_v[pl.ds(k * 16, 16)] = zeros16
        return 0

    lax.fori_loop(0, HIST // 16, z_body, 0)
    for cp in copies:
        cp.wait()

    def r0_body(j, _):
        us = [p_v[pl.ds((j * UNROLL + k) * 16, 16)] for k in range(UNROLL)]
        bs = [jnp.right_shift(u, 17) & 0x1FFF for u in us]
        for u, b in zip(us, bs):
            plsc.addupdate_scatter(hist_v, [b], ones16, mask=u >= 0)
        return 0

    lax.fori_loop(0, NVEC // UNROLL, r0_body, 0)
    b0, gt0 = _scan_hist(hist_v, HIST, K)

    t0 = b0 << 17

    def cc_body(j, cnt):
        bases = [(j * UNROLL + k) * 16 for k in range(UNROLL)]
        us = [p_v[pl.ds(base, 16)] for base in bases]
        ms = [u >= t0 for u in us]
        for base, u, m in zip(bases, us, ms):
            off = jnp.minimum(cnt, CAP)
            plsc.store_compressed(coarse_u.at[pl.ds(off, 16)], u, mask=m)
            plsc.store_compressed(coarse_idx.at[pl.ds(off, 16)], lanes + base,
                                  mask=m)
            cnt = cnt + jnp.min(plsc.all_reduce_population_count(m))
        return cnt

    ccnt = lax.fori_loop(0, NVEC // UNROLL, cc_body, jnp.int32(0))
    ccnt_c = jnp.minimum(ccnt, jnp.int32(CAP))
    coarse_u[pl.ds(ccnt_c, 16)] = neg1
    nvc = jnp.right_shift(ccnt_c + 15, 4)

    for i in range(32):
        hist_v[pl.ds(i * 16, 16)] = zeros16
    need1 = K - gt0

    def r1_body(j, _):
        u = coarse_u[pl.ds(j * 16, 16)]
        pm = jnp.right_shift(u, 17) == b0
        d = jnp.right_shift(u, 8) & 0x1FF
        plsc.addupdate_scatter(hist_v, [d], ones16, mask=pm)
        return 0

    lax.fori_loop(0, nvc, r1_body, 0)
    b1, gt1 = _scan_hist(hist_v, 512, need1)
    p2 = (b0 << 9) | b1

    for i in range(16):
        hist_v[pl.ds(i * 16, 16)] = zeros16
    need2 = need1 - gt1

    def r2_body(j, _):
        u = coarse_u[pl.ds(j * 16, 16)]
        pm = jnp.right_shift(u, 8) == p2
        d = u & 0xFF
        plsc.addupdate_scatter(hist_v, [d], ones16, mask=pm)
        return 0

    lax.fori_loop(0, nvc, r2_body, 0)
    b2, gt2 = _scan_hist(hist_v, 256, need2)
    t = (p2 << 8) | b2
    cnt_gt = gt0 + gt1 + gt2

    idxpad16 = jnp.full((16,), IDXPAD, jnp.int32)
    for j in range((KPAD + 16) // 16):
        cand_u[pl.ds(j * 16, 16)] = zeros16
        cand_idx[pl.ds(j * 16, 16)] = idxpad16

    def c_body(j, carry):
        sel_cnt, eq_cnt = carry
        u = coarse_u[pl.ds(j * 16, 16)]
        raw = coarse_idx[pl.ds(j * 16, 16)]
        cc = raw // QP
        ridx = (raw - cc * QP) * C + cc
        msel = u > t
        meq = u == t
        off1 = jnp.minimum(sel_cnt, jnp.int32(KPAD))
        plsc.store_compressed(cand_u.at[pl.ds(off1, 16)], u, mask=msel)
        plsc.store_compressed(cand_idx.at[pl.ds(off1, 16)], ridx, mask=msel)
        sel_cnt = sel_cnt + jnp.min(plsc.all_reduce_population_count(msel))
        off2 = jnp.minimum(cnt_gt + eq_cnt, jnp.int32(KPAD))
        plsc.store_compressed(cand_u.at[pl.ds(off2, 16)], u, mask=meq)
        plsc.store_compressed(cand_idx.at[pl.ds(off2, 16)], ridx, mask=meq)
        eq_cnt = eq_cnt + jnp.min(plsc.all_reduce_population_count(meq))
        return sel_cnt, eq_cnt

    lax.fori_loop(0, nvc, c_body, (jnp.int32(0), jnp.int32(0)))

    NCAND = (KPAD + 16) // 16

    def radix_pass(digit_fn, nbuckets, su, si, du, di):
        for i in range(nbuckets // 16):
            hist_v[pl.ds(i * 16, 16)] = zeros16
        for j in range(NCAND):
            d = digit_fn(su[pl.ds(j * 16, 16)], si[pl.ds(j * 16, 16)])
            cnt, last = plsc.scan_count(d)
            plsc.addupdate_scatter(hist_v, [d], cnt, mask=last)
        run = jnp.int32(0)
        for i in range(nbuckets // 16):
            cvec = hist_v[pl.ds(i * 16, 16)]
            cs = plsc.cumsum(cvec)
            hist_v[pl.ds(i * 16, 16)] = cs - cvec + run
            run = run + jnp.max(cs)
        for j in range(NCAND):
            u = su[pl.ds(j * 16, 16)]
            iv = si[pl.ds(j * 16, 16)]
            d = digit_fn(u, iv)
            cnt, last = plsc.scan_count(d)
            base = plsc.load_gather(hist_v, [d])
            pos = base + cnt - 1
            plsc.store_scatter(du, [pos], u)
            plsc.store_scatter(di, [pos], iv)
            plsc.addupdate_scatter(hist_v, [d], cnt, mask=last)

    a = (cand_u, cand_idx)
    bb = (cand2_u, cand2_idx)
    passes = [
        (lambda u, i: i & 0x1FF, 512),
        (lambda u, i: jnp.right_shift(i, 9) & 0xFF, 256),
        (lambda u, i: (~u) & 0xFF, 256),
        (lambda u, i: jnp.right_shift(~u, 8) & 0xFF, 256),
        (lambda u, i: jnp.right_shift(~u, 16) & 0xFF, 256),
        (lambda u, i: jnp.right_shift(~u, 24) & 0xFF, 256),
    ]
    for fn, nb in passes:
        radix_pass(fn, nb, a[0], a[1], bb[0], bb[1])
        a, bb = bb, a

    wv = scale_v[pl.ds(0, 16)]
    hv = scale_v[pl.ds(16, 16)]
    half = jnp.float32(0.5)
    for jj in range(KPAD // 16):
        score = cand_u[pl.ds(jj * 16, 16)]
        ridx = cand_idx[pl.ds(jj * 16, 16)]
        qfull = ridx // C
        q = jnp.minimum(qfull, Q - 1)
        lab = ridx - qfull * C
        cx = plsc.load_gather(boxes_v, [q])
        cy = plsc.load_gather(boxes_v, [q + Q])
        w = plsc.load_gather(boxes_v, [q + 2 * Q])
        h = plsc.load_gather(boxes_v, [q + 3 * Q])
        hw = half * w
        hh = half * h
        scores_v[pl.ds(jj * 16, 16)] = score
        labels_v[pl.ds(jj * 16, 16)] = lab
        boxes_ov[pl.ds(0 * KPAD + jj * 16, 16)] = (cx - hw) * wv
        boxes_ov[pl.ds(1 * KPAD + jj * 16, 16)] = (cy - hh) * hv
        boxes_ov[pl.ds(2 * KPAD + jj * 16, 16)] = (cx + hw) * wv
        boxes_ov[pl.ds(3 * KPAD + jj * 16, 16)] = (cy + hh) * hv

    pltpu.sync_copy(scores_v, scores_hbm.at[wid])
    pltpu.sync_copy(labels_v, labels_hbm.at[wid])
    pltpu.sync_copy(boxes_ov, boxes_out_hbm.at[wid])


def _sc_topk(p, boxes_flat, scale):
    mesh = plsc.VectorSubcoreMesh(core_axis_name="c", subcore_axis_name="s")
    f = pl.kernel(
        _sc_body,
        out_type=(
            jax.ShapeDtypeStruct((B, KPAD), jnp.int32),
            jax.ShapeDtypeStruct((B, KPAD), jnp.int32),
            jax.ShapeDtypeStruct((B, 4 * KPAD), jnp.float32),
        ),
        mesh=mesh,
        compiler_params=pltpu.CompilerParams(needs_layout_passes=False),
        scratch_types=[
            pltpu.VMEM((NPV,), jnp.int32),
            pltpu.VMEM((4 * Q,), jnp.float32),
            pltpu.VMEM((32,), jnp.float32),
            pltpu.VMEM((HIST,), jnp.int32),
            pltpu.VMEM((CAP + 32,), jnp.int32),
            pltpu.VMEM((CAP + 32,), jnp.int32),
            pltpu.VMEM((KPAD + 16,), jnp.int32),
            pltpu.VMEM((KPAD + 16,), jnp.int32),
            pltpu.VMEM((KPAD + 16,), jnp.int32),
            pltpu.VMEM((KPAD + 16,), jnp.int32),
            pltpu.VMEM((KPAD,), jnp.int32),
            pltpu.VMEM((KPAD,), jnp.int32),
            pltpu.VMEM((4 * KPAD,), jnp.float32),
            pltpu.SemaphoreType.DMA,
        ],
    )
    return f(p, boxes_flat, scale)


def kernel(pred_logits, pred_boxes, target_sizes):
    x_t = jnp.transpose(pred_logits, (2, 0, 1))
    p = pl.pallas_call(
        _sig_body,
        out_shape=jax.ShapeDtypeStruct((C, B, QP), jnp.int32),
    )(x_t)
    p_flat = p.reshape(C * B * QP)
    boxes_cm = jnp.transpose(pred_boxes, (0, 2, 1)).reshape(B, 4 * Q)
    ts = target_sizes.astype(jnp.float32)
    img_h = ts[:, 0]
    img_w = ts[:, 1]
    scale = jnp.concatenate(
        [jnp.broadcast_to(img_w[:, None], (B, 16)),
         jnp.broadcast_to(img_h[:, None], (B, 16))], axis=1)
    scores_p, labels_p, boxes_p = _sc_topk(p_flat, boxes_cm, scale)
    scores = jax.lax.bitcast_convert_type(scores_p[:, :K], jnp.float32)
    labels = labels_p[:, :K]
    boxes = boxes_p.reshape(B, 4, KPAD).transpose(0, 2, 1)[:, :K, :]
    return (scores, labels, boxes)
